# baseline (speedup 1.0000x reference)
import jax
import jax.numpy as jnp
from jax import lax
from jax.experimental import pallas as pl
from jax.experimental.pallas import tpu as pltpu
from jax.experimental.pallas import tpu_sc as plsc

N = 10000
E = 320000
F = 16
NC, NS = 2, 16
NW = NC * NS
NSLICE = 5
ES = E // NSLICE
EPW = ES // NW
CH = 128
GK = 5
NP = N // 8
ESP = ES // 8

_f32 = jnp.float32

_CHUNKS = [CH] * (EPW // CH)
if EPW % CH:
    _CHUNKS.append(EPW % CH)
_GROUPS = [_CHUNKS[i:i + GK] for i in range(0, len(_CHUNKS), GK)]
_GROWS = max(sum(g) for g in _GROUPS)


def _mesh():
    return plsc.VectorSubcoreMesh(core_axis_name="c", subcore_axis_name="s")


_SC_PARAMS = pltpu.CompilerParams(use_tc_tiling_on_sc=False)


def _gather_body(h_hbm, src_hbm, out_hbm, idx_all, rows0, rows1,
                 sem0, sem1):
    wid = lax.axis_index("s") * NC + lax.axis_index("c")
    base = wid * EPW
    pltpu.sync_copy(src_hbm.at[pl.ds(base, EPW)], idx_all)

    rows = (rows0, rows1)
    sems = (sem0, sem1)
    ngroups = len(_GROUPS)
    goff = [sum(sum(g) for g in _GROUPS[:i]) for i in range(ngroups)]
    pending = [None, None]

    def fire(g):
        p = g % 2
        descs = []
        off = 0
        for csz in _GROUPS[g]:
            descs.append(pltpu.async_copy(
                h_hbm.at[idx_all.at[pl.ds(goff[g] + off, csz)]],
                rows[p].at[pl.ds(off, csz)],
                sems[p]))
            off += csz
        pending[p] = descs

    def drain(g):
        p = g % 2
        for d in pending[p]:
            d.wait()
        grows = sum(_GROUPS[g])
        pltpu.sync_copy(rows[p].at[pl.ds(0, grows)],
                        out_hbm.at[pl.ds(base + goff[g], grows)])

    fire(0)
    for g in range(1, ngroups):
        fire(g)
        drain(g - 1)
    drain(ngroups - 1)


def _sc_gather(h_tab, src_s):
    fn = pl.kernel(
        _gather_body,
        out_type=jax.ShapeDtypeStruct((ES, F), _f32),
        mesh=_mesh(),
        scratch_types=[
            pltpu.VMEM((EPW,), jnp.int32),
            pltpu.VMEM((_GROWS, F), _f32),
            pltpu.VMEM((_GROWS, F), _f32),
            pltpu.SemaphoreType.DMA,
            pltpu.SemaphoreType.DMA,
        ],
        compiler_params=_SC_PARAMS,
    )
    return fn(h_tab, src_s)


_ROWS_PW = N // NS
_LASTC = _CHUNKS[-1]


def _scatter_body(m_hbm, dst_hbm, zeros_hbm, out_hbm,
                  idx0, idx1, m0, m1, idxt, mt, shared, sem0, sem1, semt):
    c = lax.axis_index("c")
    s = lax.axis_index("s")
    wid = s * NC + c
    base = wid * EPW
    nfull = len(_CHUNKS) - (1 if _LASTC != CH else 0)

    pltpu.sync_copy(zeros_hbm.at[pl.ds(s * _ROWS_PW, _ROWS_PW)],
                    shared.at[pl.ds(s * _ROWS_PW, _ROWS_PW)])
    plsc.subcore_barrier()

    idxb = (idx0, idx1)
    mb = (m0, m1)
    sems = (sem0, sem1)
    pending = [None, None]

    def fire(j):
        p = j % 2
        d1 = pltpu.async_copy(dst_hbm.at[pl.ds(base + j * CH, CH)],
                              idxb[p], sems[p])
        d2 = pltpu.async_copy(m_hbm.at[pl.ds(base + j * CH, CH)],
                              mb[p], sems[p])
        pending[p] = (d1, d2)

    fire(0)
    for j in range(nfull):
        p = j % 2
        if j + 1 < nfull:
            fire(j + 1)
        for d in pending[p]:
            d.wait()
        pltpu.sync_copy(mb[p], shared.at[idxb[p]], add=True)
    if _LASTC != CH:
        d1 = pltpu.async_copy(dst_hbm.at[pl.ds(base + nfull * CH, _LASTC)],
                              idxt, semt)
        d2 = pltpu.async_copy(m_hbm.at[pl.ds(base + nfull * CH, _LASTC)],
                              mt, semt)
        d1.wait()
        d2.wait()
        pltpu.sync_copy(mt, shared.at[idxt], add=True)

    plsc.subcore_barrier()
    pltpu.sync_copy(shared.at[pl.ds(s * _ROWS_PW, _ROWS_PW)],
                    out_hbm.at[pl.ds(c * N + s * _ROWS_PW, _ROWS_PW)])


def _sc_scatter(m_s, dst_s, zeros):
    fn = pl.kernel(
        _scatter_body,
        out_type=jax.ShapeDtypeStruct((NC * N, F), _f32),
        mesh=_mesh(),
        scratch_types=[
            pltpu.VMEM((CH,), jnp.int32),
            pltpu.VMEM((CH,), jnp.int32),
            pltpu.VMEM((CH, F), _f32),
            pltpu.VMEM((CH, F), _f32),
            pltpu.VMEM((_LASTC,), jnp.int32),
            pltpu.VMEM((_LASTC, F), _f32),
            pltpu.VMEM_SHARED((N, F), _f32),
            pltpu.SemaphoreType.DMA,
            pltpu.SemaphoreType.DMA,
            pltpu.SemaphoreType.DMA,
        ],
        compiler_params=_SC_PARAMS,
    )
    return fn(m_s, dst_s, zeros)


_BM = 8000
_BMP = _BM // 8


def _msg_body(ef_ref, hs_ref, wz_ref, bb_ref, r_ref, t_ref, m_ref):
    ef = ef_ref[...]
    hs = hs_ref[...]
    ef_rep = jnp.dot(ef, r_ref[...], preferred_element_type=_f32)
    hs_tile = jnp.dot(hs, t_ref[...], preferred_element_type=_f32)
    z = ef_rep * hs_tile
    m_ref[...] = (jnp.dot(z, wz_ref[...], preferred_element_type=_f32)
                  + jnp.dot(hs, bb_ref[...], preferred_element_type=_f32))


def _tc_messages(efeat128_s, h_src128_s, WzB, BbB, RexpB, TexpB):
    grid = (ES // _BM,)
    return pl.pallas_call(
        _msg_body,
        grid=grid,
        in_specs=[
            pl.BlockSpec((_BMP, 128), lambda i: (i, 0)),
            pl.BlockSpec((_BMP, 128), lambda i: (i, 0)),
            pl.BlockSpec((8 * F * F, 128), lambda i: (0, 0)),
            pl.BlockSpec((128, 128), lambda i: (0, 0)),
            pl.BlockSpec((128, 8 * F * F), lambda i: (0, 0)),
            pl.BlockSpec((128, 8 * F * F), lambda i: (0, 0)),
        ],
        out_specs=pl.BlockSpec((_BMP, 128), lambda i: (i, 0)),
        out_shape=jax.ShapeDtypeStruct((ESP, 128), _f32),
    )(efeat128_s, h_src128_s, WzB, BbB, RexpB, TexpB)


def _gru_body(r0_ref, r1_ref, r2_ref, r3_ref, r4_ref, h_ref,
              wi_ref, wh_ref, bi_ref, bh_ref, out_ref):
    x = (r0_ref[0:NP, :] + r0_ref[NP:2 * NP, :]
         + r1_ref[0:NP, :] + r1_ref[NP:2 * NP, :]
         + r2_ref[0:NP, :] + r2_ref[NP:2 * NP, :]
         + r3_ref[0:NP, :] + r3_ref[NP:2 * NP, :]
         + r4_ref[0:NP, :] + r4_ref[NP:2 * NP, :])
    h = h_ref[...]
    gi = jnp.dot(x, wi_ref[...], preferred_element_type=_f32) + bi_ref[...]
    gh = jnp.dot(h, wh_ref[...], preferred_element_type=_f32) + bh_ref[...]
    r = jax.nn.sigmoid(gi[:, 0:128] + gh[:, 0:128])
    z = jax.nn.sigmoid(gi[:, 128:256] + gh[:, 128:256])
    n = jnp.tanh(gi[:, 256:384] + r * gh[:, 256:384])
    out_ref[...] = (1.0 - z) * n + z * h


def _tc_gru(rsts, h128, W3i, W3h, b3i, b3h):
    return pl.pallas_call(
        _gru_body,
        out_shape=jax.ShapeDtypeStruct((NP, 128), _f32),
    )(rsts[0], rsts[1], rsts[2], rsts[3], rsts[4],
      h128, W3i, W3h, b3i, b3h)


def _pack_gru_w(WT):
    eye8 = jnp.eye(8, dtype=_f32)
    blocks = [jnp.kron(eye8, WT[:, g * F:(g + 1) * F]) for g in range(3)]
    return jnp.concatenate(blocks, axis=1)


def _pack_gru_b(b):
    return jnp.concatenate(
        [jnp.tile(b[g * F:(g + 1) * F], 8) for g in range(3)])[None, :]


@jax.jit
def kernel(feat, edge_index, efeat, W_edge, b_edge, W_ih, W_hh, b_ih, b_hh):
    src = edge_index[0]
    dst = edge_index[1]
    eye8 = jnp.eye(8, dtype=_f32)
    Wz = W_edge.reshape(F * F, F)
    Bb = b_edge.reshape(F, F)
    Rexp = jnp.repeat(jnp.eye(F, dtype=_f32), F, axis=1)
    Texp = jnp.tile(jnp.eye(F, dtype=_f32), (1, F))
    WzB = jnp.kron(eye8, Wz)
    BbB = jnp.kron(eye8, Bb)
    RexpB = jnp.kron(eye8, Rexp)
    TexpB = jnp.kron(eye8, Texp)
    W3i = _pack_gru_w(W_ih.T)
    W3h = _pack_gru_w(W_hh.T)
    b3i = _pack_gru_b(b_ih)
    b3h = _pack_gru_b(b_hh)
    zeros = jnp.zeros((N, F), _f32)
    srcs = [src[k * ES:(k + 1) * ES] for k in range(NSLICE)]
    dsts = [dst[k * ES:(k + 1) * ES] for k in range(NSLICE)]
    ef128s = [efeat[k * ES:(k + 1) * ES].reshape(ESP, 128)
              for k in range(NSLICE)]

    h128 = feat.reshape(NP, 128)
    for _ in range(2):
        h_tab = h128.reshape(N, F)
        h_srcs = [_sc_gather(h_tab, srcs[k]) for k in range(NSLICE)]
        ms = [_tc_messages(ef128s[k], h_srcs[k].reshape(ESP, 128),
                           WzB, BbB, RexpB, TexpB) for k in range(NSLICE)]
        rsts = [_sc_scatter(ms[k].reshape(ES, F), dsts[k], zeros)
                for k in range(NSLICE)]
        h128 = _tc_gru([r.reshape(2 * NP, 128) for r in rsts],
                       h128, W3i, W3h, b3i, b3h)
    return h128.reshape(N, F)


# speedup vs baseline: 1.1697x; 1.1697x over previous
import jax
import jax.numpy as jnp
from jax import lax
from jax.experimental import pallas as pl
from jax.experimental.pallas import tpu as pltpu
from jax.experimental.pallas import tpu_sc as plsc

N = 10000
E = 320000
F = 16
NC, NS = 2, 16
NW = NC * NS
NSLICE = 5
ES = E // NSLICE
EPW = ES // NW
CH = 128
GK = 5
NP = N // 8
ESP = ES // 8

_f32 = jnp.float32

_CHUNKS = [CH] * (EPW // CH)
if EPW % CH:
    _CHUNKS.append(EPW % CH)
_GROUPS = [_CHUNKS[i:i + GK] for i in range(0, len(_CHUNKS), GK)]
_GROWS = max(sum(g) for g in _GROUPS)


def _mesh():
    return plsc.VectorSubcoreMesh(core_axis_name="c", subcore_axis_name="s")


_SC_PARAMS = pltpu.CompilerParams(use_tc_tiling_on_sc=False)


def _gather_body(h_hbm, src_hbm, out_hbm, idx_all, rows0, rows1,
                 sem0, sem1, *, kbase):
    wid = lax.axis_index("s") * NC + lax.axis_index("c")
    base = kbase + wid * EPW
    obase = wid * EPW
    pltpu.sync_copy(src_hbm.at[pl.ds(base, EPW)], idx_all)

    rows = (rows0, rows1)
    sems = (sem0, sem1)
    ngroups = len(_GROUPS)
    goff = [sum(sum(g) for g in _GROUPS[:i]) for i in range(ngroups)]
    pending = [None, None]

    def fire(g):
        p = g % 2
        descs = []
        off = 0
        for csz in _GROUPS[g]:
            descs.append(pltpu.async_copy(
                h_hbm.at[idx_all.at[pl.ds(goff[g] + off, csz)]],
                rows[p].at[pl.ds(off, csz)],
                sems[p]))
            off += csz
        pending[p] = descs

    def drain(g):
        p = g % 2
        for d in pending[p]:
            d.wait()
        grows = sum(_GROUPS[g])
        pltpu.sync_copy(rows[p].at[pl.ds(0, grows)],
                        out_hbm.at[pl.ds(obase + goff[g], grows)])

    fire(0)
    for g in range(1, ngroups):
        fire(g)
        drain(g - 1)
    drain(ngroups - 1)


def _sc_gather(h_tab, src, k):
    import functools
    fn = pl.kernel(
        functools.partial(_gather_body, kbase=k * ES),
        out_type=jax.ShapeDtypeStruct((ES, F), _f32),
        mesh=_mesh(),
        scratch_types=[
            pltpu.VMEM((EPW,), jnp.int32),
            pltpu.VMEM((_GROWS, F), _f32),
            pltpu.VMEM((_GROWS, F), _f32),
            pltpu.SemaphoreType.DMA,
            pltpu.SemaphoreType.DMA,
        ],
        compiler_params=_SC_PARAMS,
    )
    return fn(h_tab, src)


_ROWS_PW = N // NS
_LASTC = _CHUNKS[-1]


def _scatter_body(m_hbm, dst_hbm, zeros_hbm, out_hbm,
                  idx0, idx1, m0, m1, idxt, mt, shared, sem0, sem1, semt,
                  *, kbase):
    c = lax.axis_index("c")
    s = lax.axis_index("s")
    wid = s * NC + c
    base = wid * EPW
    dbase = kbase + base
    nfull = len(_CHUNKS) - (1 if _LASTC != CH else 0)

    pltpu.sync_copy(zeros_hbm.at[pl.ds(s * _ROWS_PW, _ROWS_PW)],
                    shared.at[pl.ds(s * _ROWS_PW, _ROWS_PW)])
    plsc.subcore_barrier()

    idxb = (idx0, idx1)
    mb = (m0, m1)
    sems = (sem0, sem1)
    pending = [None, None]

    def fire(j):
        p = j % 2
        d1 = pltpu.async_copy(dst_hbm.at[pl.ds(dbase + j * CH, CH)],
                              idxb[p], sems[p])
        d2 = pltpu.async_copy(m_hbm.at[pl.ds(base + j * CH, CH)],
                              mb[p], sems[p])
        pending[p] = (d1, d2)

    fire(0)
    for j in range(nfull):
        p = j % 2
        if j + 1 < nfull:
            fire(j + 1)
        for d in pending[p]:
            d.wait()
        pltpu.sync_copy(mb[p], shared.at[idxb[p]], add=True)
    if _LASTC != CH:
        d1 = pltpu.async_copy(dst_hbm.at[pl.ds(dbase + nfull * CH, _LASTC)],
                              idxt, semt)
        d2 = pltpu.async_copy(m_hbm.at[pl.ds(base + nfull * CH, _LASTC)],
                              mt, semt)
        d1.wait()
        d2.wait()
        pltpu.sync_copy(mt, shared.at[idxt], add=True)

    plsc.subcore_barrier()
    pltpu.sync_copy(shared.at[pl.ds(s * _ROWS_PW, _ROWS_PW)],
                    out_hbm.at[pl.ds(c * N + s * _ROWS_PW, _ROWS_PW)])


def _sc_scatter(m_s, dst, zeros, k):
    import functools
    fn = pl.kernel(
        functools.partial(_scatter_body, kbase=k * ES),
        out_type=jax.ShapeDtypeStruct((NC * N, F), _f32),
        mesh=_mesh(),
        scratch_types=[
            pltpu.VMEM((CH,), jnp.int32),
            pltpu.VMEM((CH,), jnp.int32),
            pltpu.VMEM((CH, F), _f32),
            pltpu.VMEM((CH, F), _f32),
            pltpu.VMEM((_LASTC,), jnp.int32),
            pltpu.VMEM((_LASTC, F), _f32),
            pltpu.VMEM_SHARED((N, F), _f32),
            pltpu.SemaphoreType.DMA,
            pltpu.SemaphoreType.DMA,
            pltpu.SemaphoreType.DMA,
        ],
        compiler_params=_SC_PARAMS,
    )
    return fn(m_s, dst, zeros)


_BM = 8000
_BMP = _BM // 8


def _msg_body(ef_ref, hs_ref, wz_ref, bb_ref, r_ref, t_ref, m_ref):
    ef = ef_ref[...]
    hs = hs_ref[...]
    ef_rep = jnp.dot(ef, r_ref[...], preferred_element_type=_f32)
    hs_tile = jnp.dot(hs, t_ref[...], preferred_element_type=_f32)
    z = ef_rep * hs_tile
    m_ref[...] = (jnp.dot(z, wz_ref[...], preferred_element_type=_f32)
                  + jnp.dot(hs, bb_ref[...], preferred_element_type=_f32))


def _tc_messages(efeat128, h_src128_s, WzB, BbB, RexpB, TexpB, k):
    grid = (ES // _BM,)
    koff = k * (ESP // _BMP)
    return pl.pallas_call(
        _msg_body,
        grid=grid,
        in_specs=[
            pl.BlockSpec((_BMP, 128), lambda i: (i + koff, 0)),
            pl.BlockSpec((_BMP, 128), lambda i: (i, 0)),
            pl.BlockSpec((8 * F * F, 128), lambda i: (0, 0)),
            pl.BlockSpec((128, 128), lambda i: (0, 0)),
            pl.BlockSpec((128, 8 * F * F), lambda i: (0, 0)),
            pl.BlockSpec((128, 8 * F * F), lambda i: (0, 0)),
        ],
        out_specs=pl.BlockSpec((_BMP, 128), lambda i: (i, 0)),
        out_shape=jax.ShapeDtypeStruct((ESP, 128), _f32),
    )(efeat128, h_src128_s, WzB, BbB, RexpB, TexpB)


def _gru_body(r0_ref, r1_ref, r2_ref, r3_ref, r4_ref, h_ref,
              wi_ref, wh_ref, bi_ref, bh_ref, out_ref):
    x = (r0_ref[0:NP, :] + r0_ref[NP:2 * NP, :]
         + r1_ref[0:NP, :] + r1_ref[NP:2 * NP, :]
         + r2_ref[0:NP, :] + r2_ref[NP:2 * NP, :]
         + r3_ref[0:NP, :] + r3_ref[NP:2 * NP, :]
         + r4_ref[0:NP, :] + r4_ref[NP:2 * NP, :])
    h = h_ref[...]
    gi = jnp.dot(x, wi_ref[...], preferred_element_type=_f32) + bi_ref[...]
    gh = jnp.dot(h, wh_ref[...], preferred_element_type=_f32) + bh_ref[...]
    r = jax.nn.sigmoid(gi[:, 0:128] + gh[:, 0:128])
    z = jax.nn.sigmoid(gi[:, 128:256] + gh[:, 128:256])
    n = jnp.tanh(gi[:, 256:384] + r * gh[:, 256:384])
    out_ref[...] = (1.0 - z) * n + z * h


def _tc_gru(rsts, h128, W3i, W3h, b3i, b3h):
    return pl.pallas_call(
        _gru_body,
        out_shape=jax.ShapeDtypeStruct((NP, 128), _f32),
    )(rsts[0], rsts[1], rsts[2], rsts[3], rsts[4],
      h128, W3i, W3h, b3i, b3h)


def _pack_gru_w(WT):
    eye8 = jnp.eye(8, dtype=_f32)
    blocks = [jnp.kron(eye8, WT[:, g * F:(g + 1) * F]) for g in range(3)]
    return jnp.concatenate(blocks, axis=1)


def _pack_gru_b(b):
    return jnp.concatenate(
        [jnp.tile(b[g * F:(g + 1) * F], 8) for g in range(3)])[None, :]


@jax.jit
def kernel(feat, edge_index, efeat, W_edge, b_edge, W_ih, W_hh, b_ih, b_hh):
    src = edge_index[0]
    dst = edge_index[1]
    eye8 = jnp.eye(8, dtype=_f32)
    Wz = W_edge.reshape(F * F, F)
    Bb = b_edge.reshape(F, F)
    Rexp = jnp.repeat(jnp.eye(F, dtype=_f32), F, axis=1)
    Texp = jnp.tile(jnp.eye(F, dtype=_f32), (1, F))
    WzB = jnp.kron(eye8, Wz)
    BbB = jnp.kron(eye8, Bb)
    RexpB = jnp.kron(eye8, Rexp)
    TexpB = jnp.kron(eye8, Texp)
    W3i = _pack_gru_w(W_ih.T)
    W3h = _pack_gru_w(W_hh.T)
    b3i = _pack_gru_b(b_ih)
    b3h = _pack_gru_b(b_hh)
    zeros = jnp.zeros((N, F), _f32)
    efeat128 = efeat.reshape(E // 8, 128)

    h128 = feat.reshape(NP, 128)
    for _ in range(2):
        h_tab = h128.reshape(N, F)
        h_srcs = [_sc_gather(h_tab, src, k) for k in range(NSLICE)]
        ms = [_tc_messages(efeat128, h_srcs[k].reshape(ESP, 128),
                           WzB, BbB, RexpB, TexpB, k) for k in range(NSLICE)]
        rsts = [_sc_scatter(ms[k].reshape(ES, F), dst, zeros, k)
                for k in range(NSLICE)]
        h128 = _tc_gru([r.reshape(2 * NP, 128) for r in rsts],
                       h128, W3i, W3h, b3i, b3h)
    return h128.reshape(N, F)
